# initial kernel scaffold (unmeasured)
import jax
import jax.numpy as jnp
from jax import lax
from jax.experimental import pallas as pl
from jax.experimental.pallas import tpu as pltpu

N_DEV = 8


def kernel(x, w_mat):
    m, k_per = x.shape
    _, n = w_mat.shape
    m_per = m // N_DEV

    def body(x_ref, w_ref, out_ref, comm_ref, amax_ref,
             send_sems, recv_sems, amax_send_sems, amax_recv_sems, credit_sem):
        my = lax.axis_index("i")
        left = lax.rem(my + N_DEV - 1, N_DEV)
        right = lax.rem(my + 1, N_DEV)

        barrier_sem = pltpu.get_barrier_semaphore()
        for nbr in (left, right):
            pl.semaphore_signal(barrier_sem, inc=1, device_id=(nbr,),
                                device_id_type=pl.DeviceIdType.MESH)
        pl.semaphore_wait(barrier_sem, 2)

        def partial_chunk(c):
            xs = x_ref[pl.ds(c * m_per, m_per), :]
            return jnp.dot(xs, w_ref[:, :], preferred_element_type=jnp.float32)

        comm_ref[0, :, :] = partial_chunk(lax.rem(my + N_DEV - 1, N_DEV))

        for s in range(1, N_DEV):
            send_slot = (s - 1) % 2
            recv_slot = s % 2
            if s >= 2:
                pl.semaphore_wait(credit_sem, 1)
            rdma = pltpu.make_async_remote_copy(
                src_ref=comm_ref.at[send_slot],
                dst_ref=comm_ref.at[recv_slot],
                send_sem=send_sems.at[send_slot],
                recv_sem=recv_sems.at[recv_slot],
                device_id=(right,),
                device_id_type=pl.DeviceIdType.MESH,
            )
            rdma.start()
            rdma.wait()
            pl.semaphore_signal(credit_sem, inc=1, device_id=(left,),
                                device_id_type=pl.DeviceIdType.MESH)
            c = lax.rem(my + N_DEV - 1 - s, N_DEV)
            comm_ref[recv_slot, :, :] = comm_ref[recv_slot, :, :] + partial_chunk(c)

        comm_ref[0, :, :] = jnp.maximum(comm_ref[1, :, :], 0.0)
        local_amax = jnp.max(comm_ref[0, :, :])
        amax_ref[pl.ds(0, 1), :] = jnp.full((1, 128), local_amax, jnp.float32)

        copies = []
        for o in range(1, N_DEV):
            tgt = lax.rem(my + o, N_DEV)
            rd = pltpu.make_async_remote_copy(
                src_ref=amax_ref.at[pl.ds(0, 1)],
                dst_ref=amax_ref.at[pl.ds(o, 1)],
                send_sem=amax_send_sems.at[o],
                recv_sem=amax_recv_sems.at[o],
                device_id=(tgt,),
                device_id_type=pl.DeviceIdType.MESH,
            )
            rd.start()
            copies.append(rd)
        for rd in copies:
            rd.wait()

        gmax = jnp.max(amax_ref[:, :])
        scale = gmax / 127.0
        q = jnp.clip(jnp.round(comm_ref[0, :, :] / scale), -127.0, 127.0)
        out_ref[:, :] = q * scale

    return pl.pallas_call(
        body,
        out_shape=jax.ShapeDtypeStruct((m_per, n), jnp.float32),
        in_specs=[pl.BlockSpec(memory_space=pltpu.VMEM),
                  pl.BlockSpec(memory_space=pltpu.VMEM)],
        out_specs=pl.BlockSpec(memory_space=pltpu.VMEM),
        scratch_shapes=[
            pltpu.VMEM((2, m_per, n), jnp.float32),
            pltpu.VMEM((N_DEV, 128), jnp.float32),
            pltpu.SemaphoreType.DMA((2,)),
            pltpu.SemaphoreType.DMA((2,)),
            pltpu.SemaphoreType.DMA((N_DEV,)),
            pltpu.SemaphoreType.DMA((N_DEV,)),
            pltpu.SemaphoreType.REGULAR,
        ],
        compiler_params=pltpu.CompilerParams(collective_id=0),
    )(x, w_mat)


# baseline (device time: 1379468 ns/iter reference)
import jax
import jax.numpy as jnp
from jax import lax
from jax.experimental import pallas as pl
from jax.experimental.pallas import tpu as pltpu

N_DEV = 8
NP = 2


def kernel(x, w_mat):
    m, k_per = x.shape
    _, n = w_mat.shape
    m_per = m // N_DEV
    n_p = n // NP

    def body(x_ref, w_ref, out_ref, comm_ref, amax_ref,
             send_sems, recv_sems, amax_send_sems, amax_recv_sems,
             credit_sem, local_sem):
        my = lax.axis_index("i")
        left = lax.rem(my + N_DEV - 1, N_DEV)
        right = lax.rem(my + 1, N_DEV)

        barrier_sem = pltpu.get_barrier_semaphore()
        for nbr in (left, right):
            pl.semaphore_signal(barrier_sem, inc=1, device_id=(nbr,),
                                device_id_type=pl.DeviceIdType.MESH)
        pl.semaphore_wait(barrier_sem, 2)

        local_amax = jnp.float32(0.0)

        for p in range(NP):
            col = p * n_p

            def partial_chunk(c, col=col):
                xs = x_ref[pl.ds(c * m_per, m_per), :]
                return jnp.dot(xs, w_ref[:, col:col + n_p],
                               preferred_element_type=jnp.float32)

            comm_ref[0, :, :] = partial_chunk(lax.rem(my + N_DEV - 1, N_DEV))

            for s in range(1, N_DEV):
                send_slot = (s - 1) % 2
                recv_slot = s % 2
                if s >= 2 or p >= 1:
                    pl.semaphore_wait(credit_sem, 1)
                rdma = pltpu.make_async_remote_copy(
                    src_ref=comm_ref.at[send_slot],
                    dst_ref=comm_ref.at[recv_slot],
                    send_sem=send_sems.at[send_slot],
                    recv_sem=recv_sems.at[recv_slot],
                    device_id=(right,),
                    device_id_type=pl.DeviceIdType.MESH,
                )
                rdma.start()
                rdma.wait()
                if s <= 6:
                    pl.semaphore_signal(credit_sem, inc=1, device_id=(left,),
                                        device_id_type=pl.DeviceIdType.MESH)
                c = lax.rem(my + N_DEV - 1 - s, N_DEV)
                comm_ref[recv_slot, :, :] = (
                    comm_ref[recv_slot, :, :] + partial_chunk(c))

            comm_ref[0, :, :] = jnp.maximum(comm_ref[1, :, :], 0.0)
            if p < NP - 1:
                pl.semaphore_signal(credit_sem, inc=1, device_id=(left,),
                                    device_id_type=pl.DeviceIdType.MESH)
            local_amax = jnp.maximum(local_amax, jnp.max(comm_ref[0, :, :]))
            stage = pltpu.make_async_copy(
                comm_ref.at[0], out_ref.at[:, pl.ds(col, n_p)], local_sem)
            stage.start()
            stage.wait()

        amax_ref[pl.ds(0, 1), :] = jnp.full((1, 128), local_amax, jnp.float32)

        copies = []
        for o in range(1, N_DEV):
            tgt = lax.rem(my + o, N_DEV)
            rd = pltpu.make_async_remote_copy(
                src_ref=amax_ref.at[pl.ds(0, 1)],
                dst_ref=amax_ref.at[pl.ds(o, 1)],
                send_sem=amax_send_sems.at[o],
                recv_sem=amax_recv_sems.at[o],
                device_id=(tgt,),
                device_id_type=pl.DeviceIdType.MESH,
            )
            rd.start()
            copies.append(rd)
        for rd in copies:
            rd.wait()

        gmax = jnp.max(amax_ref[:, :])
        scale = gmax / 127.0

        for p in range(NP):
            col = p * n_p
            load = pltpu.make_async_copy(
                out_ref.at[:, pl.ds(col, n_p)], comm_ref.at[0], local_sem)
            load.start()
            load.wait()
            q = jnp.clip(jnp.round(comm_ref[0, :, :] / scale), -127.0, 127.0)
            comm_ref[0, :, :] = q * scale
            store = pltpu.make_async_copy(
                comm_ref.at[0], out_ref.at[:, pl.ds(col, n_p)], local_sem)
            store.start()
            store.wait()

    return pl.pallas_call(
        body,
        out_shape=jax.ShapeDtypeStruct((m_per, n), jnp.float32),
        in_specs=[pl.BlockSpec(memory_space=pltpu.VMEM),
                  pl.BlockSpec(memory_space=pltpu.VMEM)],
        out_specs=pl.BlockSpec(memory_space=pl.ANY),
        scratch_shapes=[
            pltpu.VMEM((2, m_per, n_p), jnp.float32),
            pltpu.VMEM((N_DEV, 128), jnp.float32),
            pltpu.SemaphoreType.DMA((2,)),
            pltpu.SemaphoreType.DMA((2,)),
            pltpu.SemaphoreType.DMA((N_DEV,)),
            pltpu.SemaphoreType.DMA((N_DEV,)),
            pltpu.SemaphoreType.REGULAR,
            pltpu.SemaphoreType.DMA,
        ],
        compiler_params=pltpu.CompilerParams(
            collective_id=0, vmem_limit_bytes=38 * 1024 * 1024),
    )(x, w_mat)


# device time: 736635 ns/iter; 1.8727x vs baseline; 1.8727x over previous
import jax
import jax.numpy as jnp
from jax import lax
from jax.experimental import pallas as pl
from jax.experimental.pallas import tpu as pltpu

N_DEV = 8
NP = 2


def kernel(x, w_mat):
    m, k_per = x.shape
    _, n = w_mat.shape
    m_per = m // N_DEV
    n_p = n // NP
    n_h = n_p // 2

    def body(x_ref, w_ref, out_ref, comm_a, comm_b, tmp_a, tmp_b, amax_ref,
             sa_send, sa_recv, sb_send, sb_recv,
             amax_send_sems, amax_recv_sems, credit_a, credit_b, local_sems):
        my = lax.axis_index("i")
        left = lax.rem(my + N_DEV - 1, N_DEV)
        right = lax.rem(my + 1, N_DEV)

        barrier_sem = pltpu.get_barrier_semaphore()
        for nbr in (left, right):
            pl.semaphore_signal(barrier_sem, inc=1, device_id=(nbr,),
                                device_id_type=pl.DeviceIdType.MESH)
        pl.semaphore_wait(barrier_sem, 2)

        local_amax = jnp.float32(0.0)

        for p in range(NP):
            col_a = p * n_p
            col_b = p * n_p + n_h

            def part(c, col):
                xs = x_ref[pl.ds(c * m_per, m_per), :]
                return jnp.dot(xs, w_ref[:, col:col + n_h],
                               preferred_element_type=jnp.float32)

            comm_a[0, :, :] = part(lax.rem(my + N_DEV - 1, N_DEV), col_a)
            comm_b[0, :, :] = part(lax.rem(my + 1, N_DEV), col_b)

            for s in range(1, N_DEV):
                ss = (s - 1) % 2
                rs = s % 2
                if s >= 2 or p >= 1:
                    pl.semaphore_wait(credit_a, 1)
                    pl.semaphore_wait(credit_b, 1)
                ra = pltpu.make_async_remote_copy(
                    src_ref=comm_a.at[ss], dst_ref=comm_a.at[rs],
                    send_sem=sa_send.at[ss], recv_sem=sa_recv.at[rs],
                    device_id=(right,), device_id_type=pl.DeviceIdType.MESH)
                rb = pltpu.make_async_remote_copy(
                    src_ref=comm_b.at[ss], dst_ref=comm_b.at[rs],
                    send_sem=sb_send.at[ss], recv_sem=sb_recv.at[rs],
                    device_id=(left,), device_id_type=pl.DeviceIdType.MESH)
                ra.start()
                rb.start()
                tmp_a[:, :] = part(lax.rem(my + N_DEV - 1 - s, N_DEV), col_a)
                tmp_b[:, :] = part(lax.rem(my + 1 + s, N_DEV), col_b)
                ra.wait()
                rb.wait()
                if s <= 6:
                    pl.semaphore_signal(credit_a, inc=1, device_id=(left,),
                                        device_id_type=pl.DeviceIdType.MESH)
                    pl.semaphore_signal(credit_b, inc=1, device_id=(right,),
                                        device_id_type=pl.DeviceIdType.MESH)
                comm_a[rs, :, :] = comm_a[rs, :, :] + tmp_a[:, :]
                comm_b[rs, :, :] = comm_b[rs, :, :] + tmp_b[:, :]

            comm_a[0, :, :] = jnp.maximum(comm_a[1, :, :], 0.0)
            comm_b[0, :, :] = jnp.maximum(comm_b[1, :, :], 0.0)
            if p < NP - 1:
                pl.semaphore_signal(credit_a, inc=1, device_id=(left,),
                                    device_id_type=pl.DeviceIdType.MESH)
                pl.semaphore_signal(credit_b, inc=1, device_id=(right,),
                                    device_id_type=pl.DeviceIdType.MESH)
            local_amax = jnp.maximum(local_amax, jnp.max(comm_a[0, :, :]))
            local_amax = jnp.maximum(local_amax, jnp.max(comm_b[0, :, :]))
            st_a = pltpu.make_async_copy(
                comm_a.at[0], out_ref.at[:, pl.ds(col_a, n_h)],
                local_sems.at[0])
            st_b = pltpu.make_async_copy(
                comm_b.at[0], out_ref.at[:, pl.ds(col_b, n_h)],
                local_sems.at[1])
            st_a.start()
            st_b.start()
            st_a.wait()
            st_b.wait()

        amax_ref[pl.ds(0, 1), :] = jnp.full((1, 128), local_amax, jnp.float32)

        copies = []
        for o in range(1, N_DEV):
            tgt = lax.rem(my + o, N_DEV)
            rd = pltpu.make_async_remote_copy(
                src_ref=amax_ref.at[pl.ds(0, 1)],
                dst_ref=amax_ref.at[pl.ds(o, 1)],
                send_sem=amax_send_sems.at[o],
                recv_sem=amax_recv_sems.at[o],
                device_id=(tgt,),
                device_id_type=pl.DeviceIdType.MESH,
            )
            rd.start()
            copies.append(rd)
        for rd in copies:
            rd.wait()

        gmax = jnp.max(amax_ref[:, :])
        scale = gmax / 127.0

        for g in range(2 * NP):
            col = g * n_h
            buf = comm_a if g % 2 == 0 else comm_b
            sem = local_sems.at[g % 2]
            load = pltpu.make_async_copy(
                out_ref.at[:, pl.ds(col, n_h)], buf.at[0], sem)
            load.start()
            load.wait()
            q = jnp.clip(jnp.round(buf[0, :, :] / scale), -127.0, 127.0)
            buf[0, :, :] = q * scale
            store = pltpu.make_async_copy(
                buf.at[0], out_ref.at[:, pl.ds(col, n_h)], sem)
            store.start()
            store.wait()

    return pl.pallas_call(
        body,
        out_shape=jax.ShapeDtypeStruct((m_per, n), jnp.float32),
        in_specs=[pl.BlockSpec(memory_space=pltpu.VMEM),
                  pl.BlockSpec(memory_space=pltpu.VMEM)],
        out_specs=pl.BlockSpec(memory_space=pl.ANY),
        scratch_shapes=[
            pltpu.VMEM((2, m_per, n_h), jnp.float32),
            pltpu.VMEM((2, m_per, n_h), jnp.float32),
            pltpu.VMEM((m_per, n_h), jnp.float32),
            pltpu.VMEM((m_per, n_h), jnp.float32),
            pltpu.VMEM((N_DEV, 128), jnp.float32),
            pltpu.SemaphoreType.DMA((2,)),
            pltpu.SemaphoreType.DMA((2,)),
            pltpu.SemaphoreType.DMA((2,)),
            pltpu.SemaphoreType.DMA((2,)),
            pltpu.SemaphoreType.DMA((N_DEV,)),
            pltpu.SemaphoreType.DMA((N_DEV,)),
            pltpu.SemaphoreType.REGULAR,
            pltpu.SemaphoreType.REGULAR,
            pltpu.SemaphoreType.DMA((2,)),
        ],
        compiler_params=pltpu.CompilerParams(
            collective_id=0, vmem_limit_bytes=40 * 1024 * 1024),
    )(x, w_mat)


# device time: 717875 ns/iter; 1.9216x vs baseline; 1.0261x over previous
import jax
import jax.numpy as jnp
from jax import lax
from jax.experimental import pallas as pl
from jax.experimental.pallas import tpu as pltpu

N_DEV = 8
NP = 2


def kernel(x, w_mat):
    m, k_per = x.shape
    _, n = w_mat.shape
    m_per = m // N_DEV
    n_p = n // NP
    n_h = n_p // 2

    def body(x_ref, w_ref, out_ref, comm_a, comm_b, tmp_a, tmp_b, amax_ref,
             sa_send, sa_recv, sb_send, sb_recv,
             amax_send_sems, amax_recv_sems, credit_a, credit_b, local_sems):
        my = lax.axis_index("i")
        left = lax.rem(my + N_DEV - 1, N_DEV)
        right = lax.rem(my + 1, N_DEV)

        barrier_sem = pltpu.get_barrier_semaphore()
        for nbr in (left, right):
            pl.semaphore_signal(barrier_sem, inc=1, device_id=(nbr,),
                                device_id_type=pl.DeviceIdType.MESH)
        pl.semaphore_wait(barrier_sem, 2)

        n_q = n_h // 2

        def part_into(ref, c, col):
            xs = x_ref[pl.ds(c * m_per, m_per), :]
            for j in range(0, n_h, n_q):
                ref[:, pl.ds(j, n_q)] = jnp.dot(
                    xs, w_ref[:, col + j:col + j + n_q],
                    preferred_element_type=jnp.float32)

        local_amax = jnp.float32(0.0)
        st_a = st_b = None

        part_into(comm_a.at[0], lax.rem(my + N_DEV - 1, N_DEV), 0 * n_h)
        part_into(comm_b.at[0], lax.rem(my + 1, N_DEV), 1 * n_h)

        for p in range(NP):
            col_a = p * n_p
            col_b = p * n_p + n_h

            for s in range(1, N_DEV):
                ss = (s - 1) % 2
                rs = s % 2
                if s >= 2 or p >= 1:
                    pl.semaphore_wait(credit_a, 1)
                    pl.semaphore_wait(credit_b, 1)
                ra = pltpu.make_async_remote_copy(
                    src_ref=comm_a.at[ss], dst_ref=comm_a.at[rs],
                    send_sem=sa_send.at[ss], recv_sem=sa_recv.at[rs],
                    device_id=(right,), device_id_type=pl.DeviceIdType.MESH)
                rb = pltpu.make_async_remote_copy(
                    src_ref=comm_b.at[ss], dst_ref=comm_b.at[rs],
                    send_sem=sb_send.at[ss], recv_sem=sb_recv.at[rs],
                    device_id=(left,), device_id_type=pl.DeviceIdType.MESH)
                ra.start()
                rb.start()
                if s == 1 and st_a is not None:
                    st_a.wait()
                    st_b.wait()
                part_into(tmp_a, lax.rem(my + N_DEV - 1 - s, N_DEV), col_a)
                part_into(tmp_b, lax.rem(my + 1 + s, N_DEV), col_b)
                ra.wait()
                rb.wait()
                if s <= 6:
                    pl.semaphore_signal(credit_a, inc=1, device_id=(left,),
                                        device_id_type=pl.DeviceIdType.MESH)
                    pl.semaphore_signal(credit_b, inc=1, device_id=(right,),
                                        device_id_type=pl.DeviceIdType.MESH)
                for j in range(0, n_h, n_q):
                    sl = pl.ds(j, n_q)
                    comm_a[rs, :, sl] = comm_a[rs, :, sl] + tmp_a[:, sl]
                    comm_b[rs, :, sl] = comm_b[rs, :, sl] + tmp_b[:, sl]

            for j in range(0, n_h, n_q):
                sl = pl.ds(j, n_q)
                tmp_a[:, sl] = jnp.maximum(comm_a[1, :, sl], 0.0)
                tmp_b[:, sl] = jnp.maximum(comm_b[1, :, sl], 0.0)
            if p < NP - 1:
                pl.semaphore_signal(credit_a, inc=1, device_id=(left,),
                                    device_id_type=pl.DeviceIdType.MESH)
                pl.semaphore_signal(credit_b, inc=1, device_id=(right,),
                                    device_id_type=pl.DeviceIdType.MESH)
            local_amax = jnp.maximum(local_amax, jnp.max(tmp_a[:, :]))
            local_amax = jnp.maximum(local_amax, jnp.max(tmp_b[:, :]))
            if p < NP - 1:
                st_a = pltpu.make_async_copy(
                    tmp_a, out_ref.at[:, pl.ds(col_a, n_h)], local_sems.at[0])
                st_b = pltpu.make_async_copy(
                    tmp_b, out_ref.at[:, pl.ds(col_b, n_h)], local_sems.at[1])
                st_a.start()
                st_b.start()
                part_into(comm_a.at[0], lax.rem(my + N_DEV - 1, N_DEV),
                          col_a + n_p)
                part_into(comm_b.at[0], lax.rem(my + 1, N_DEV), col_b + n_p)

        amax_ref[pl.ds(0, 1), :] = jnp.full((1, 128), local_amax, jnp.float32)

        copies = []
        for o in range(1, N_DEV):
            tgt = lax.rem(my + o, N_DEV)
            rd = pltpu.make_async_remote_copy(
                src_ref=amax_ref.at[pl.ds(0, 1)],
                dst_ref=amax_ref.at[pl.ds(o, 1)],
                send_sem=amax_send_sems.at[o],
                recv_sem=amax_recv_sems.at[o],
                device_id=(tgt,),
                device_id_type=pl.DeviceIdType.MESH,
            )
            rd.start()
            copies.append(rd)

        loads = []
        for p in range(NP - 1):
            ld_a = pltpu.make_async_copy(
                out_ref.at[:, pl.ds(p * n_p, n_h)], comm_a.at[p],
                local_sems.at[0])
            ld_b = pltpu.make_async_copy(
                out_ref.at[:, pl.ds(p * n_p + n_h, n_h)], comm_b.at[p],
                local_sems.at[1])
            ld_a.start()
            ld_b.start()
            loads.append((ld_a, ld_b))

        for rd in copies:
            rd.wait()
        gmax = jnp.max(amax_ref[:, :])
        scale = gmax / 127.0

        def quant_inplace(ref, row):
            for j in range(0, n_h, n_h // 2):
                sl = pl.ds(j, n_h // 2)
                if row is None:
                    v = ref[:, sl]
                    ref[:, sl] = jnp.clip(
                        jnp.round(v / scale), -127.0, 127.0) * scale
                else:
                    v = ref[row, :, sl]
                    ref[row, :, sl] = jnp.clip(
                        jnp.round(v / scale), -127.0, 127.0) * scale

        stores = []
        for p in range(NP - 1):
            ld_a, ld_b = loads[p]
            ld_a.wait()
            quant_inplace(comm_a, p)
            so_a = pltpu.make_async_copy(
                comm_a.at[p], out_ref.at[:, pl.ds(p * n_p, n_h)],
                local_sems.at[0])
            so_a.start()
            ld_b.wait()
            quant_inplace(comm_b, p)
            so_b = pltpu.make_async_copy(
                comm_b.at[p], out_ref.at[:, pl.ds(p * n_p + n_h, n_h)],
                local_sems.at[1])
            so_b.start()
            stores.extend((so_a, so_b))
        quant_inplace(tmp_a, None)
        so_a = pltpu.make_async_copy(
            tmp_a, out_ref.at[:, pl.ds((NP - 1) * n_p, n_h)],
            local_sems.at[2])
        so_a.start()
        quant_inplace(tmp_b, None)
        so_b = pltpu.make_async_copy(
            tmp_b, out_ref.at[:, pl.ds((NP - 1) * n_p + n_h, n_h)],
            local_sems.at[3])
        so_b.start()
        stores.extend((so_a, so_b))
        for so in stores:
            so.wait()

    return pl.pallas_call(
        body,
        out_shape=jax.ShapeDtypeStruct((m_per, n), jnp.float32),
        in_specs=[pl.BlockSpec(memory_space=pltpu.VMEM),
                  pl.BlockSpec(memory_space=pltpu.VMEM)],
        out_specs=pl.BlockSpec(memory_space=pl.ANY),
        scratch_shapes=[
            pltpu.VMEM((2, m_per, n_h), jnp.float32),
            pltpu.VMEM((2, m_per, n_h), jnp.float32),
            pltpu.VMEM((m_per, n_h), jnp.float32),
            pltpu.VMEM((m_per, n_h), jnp.float32),
            pltpu.VMEM((N_DEV, 128), jnp.float32),
            pltpu.SemaphoreType.DMA((2,)),
            pltpu.SemaphoreType.DMA((2,)),
            pltpu.SemaphoreType.DMA((2,)),
            pltpu.SemaphoreType.DMA((2,)),
            pltpu.SemaphoreType.DMA((N_DEV,)),
            pltpu.SemaphoreType.DMA((N_DEV,)),
            pltpu.SemaphoreType.REGULAR,
            pltpu.SemaphoreType.REGULAR,
            pltpu.SemaphoreType.DMA((4,)),
        ],
        compiler_params=pltpu.CompilerParams(
            collective_id=0, vmem_limit_bytes=41 * 1024 * 1024),
    )(x, w_mat)


# device time: 717043 ns/iter; 1.9238x vs baseline; 1.0012x over previous
import jax
import jax.numpy as jnp
from jax import lax
from jax.experimental import pallas as pl
from jax.experimental.pallas import tpu as pltpu

N_DEV = 8
NP = 2


def kernel(x, w_mat):
    m, k_per = x.shape
    _, n = w_mat.shape
    m_per = m // N_DEV
    n_p = n // NP
    n_h = n_p // 2

    def body(x_ref, w_ref, out_ref, comm_a, comm_b, tmp_a, tmp_b, amax_ref,
             sa_send, sa_recv, sb_send, sb_recv, sub_send, sub_recv,
             amax_send_sems, amax_recv_sems, credit_a, credit_b, local_sems):
        my = lax.axis_index("i")
        left = lax.rem(my + N_DEV - 1, N_DEV)
        right = lax.rem(my + 1, N_DEV)

        barrier_sem = pltpu.get_barrier_semaphore()
        for nbr in (left, right):
            pl.semaphore_signal(barrier_sem, inc=1, device_id=(nbr,),
                                device_id_type=pl.DeviceIdType.MESH)
        pl.semaphore_wait(barrier_sem, 2)

        n_q = n_h // 2

        def part_into(ref, c, col):
            xs = x_ref[pl.ds(c * m_per, m_per), :]
            for j in range(0, n_h, n_q):
                ref[:, pl.ds(j, n_q)] = jnp.dot(
                    xs, w_ref[:, col + j:col + j + n_q],
                    preferred_element_type=jnp.float32)

        local_amax = jnp.float32(0.0)
        st_a = st_b = None

        for p in range(NP):
            col_a = p * n_p
            col_b = p * n_p + n_h

            if p >= 1:
                pl.semaphore_wait(credit_a, 1)
                pl.semaphore_wait(credit_b, 1)
            c_a = lax.rem(my + N_DEV - 1, N_DEV)
            c_b = lax.rem(my + 1, N_DEV)
            xs_a = x_ref[pl.ds(c_a * m_per, m_per), :]
            xs_b = x_ref[pl.ds(c_b * m_per, m_per), :]
            subs = []
            for j in range(2):
                sl = pl.ds(j * n_q, n_q)
                for k, (xs, col, comm, s_send, s_recv, dev) in enumerate([
                        (xs_a, col_a, comm_a, sub_send, sub_recv, right),
                        (xs_b, col_b, comm_b, sub_send, sub_recv, left)]):
                    comm[0, :, sl] = jnp.dot(
                        xs, w_ref[:, col + j * n_q:col + (j + 1) * n_q],
                        preferred_element_type=jnp.float32)
                    rd = pltpu.make_async_remote_copy(
                        src_ref=comm.at[0, :, sl], dst_ref=comm.at[1, :, sl],
                        send_sem=s_send.at[2 * j + k],
                        recv_sem=s_recv.at[2 * j + k],
                        device_id=(dev,),
                        device_id_type=pl.DeviceIdType.MESH)
                    rd.start()
                    subs.append(rd)
            if st_a is not None:
                st_a.wait()
                st_b.wait()
            part_into(tmp_a, lax.rem(my + N_DEV - 2, N_DEV), col_a)
            part_into(tmp_b, lax.rem(my + 2, N_DEV), col_b)
            for rd in subs:
                rd.wait()
            pl.semaphore_signal(credit_a, inc=1, device_id=(left,),
                                device_id_type=pl.DeviceIdType.MESH)
            pl.semaphore_signal(credit_b, inc=1, device_id=(right,),
                                device_id_type=pl.DeviceIdType.MESH)
            for j in range(0, n_h, n_q):
                sl = pl.ds(j, n_q)
                comm_a[1, :, sl] = comm_a[1, :, sl] + tmp_a[:, sl]
                comm_b[1, :, sl] = comm_b[1, :, sl] + tmp_b[:, sl]

            for s in range(2, N_DEV):
                ss = (s - 1) % 2
                rs = s % 2
                pl.semaphore_wait(credit_a, 1)
                pl.semaphore_wait(credit_b, 1)
                ra = pltpu.make_async_remote_copy(
                    src_ref=comm_a.at[ss], dst_ref=comm_a.at[rs],
                    send_sem=sa_send.at[ss], recv_sem=sa_recv.at[rs],
                    device_id=(right,), device_id_type=pl.DeviceIdType.MESH)
                rb = pltpu.make_async_remote_copy(
                    src_ref=comm_b.at[ss], dst_ref=comm_b.at[rs],
                    send_sem=sb_send.at[ss], recv_sem=sb_recv.at[rs],
                    device_id=(left,), device_id_type=pl.DeviceIdType.MESH)
                ra.start()
                rb.start()
                part_into(tmp_a, lax.rem(my + N_DEV - 1 - s, N_DEV), col_a)
                part_into(tmp_b, lax.rem(my + 1 + s, N_DEV), col_b)
                ra.wait()
                rb.wait()
                if s <= 6:
                    pl.semaphore_signal(credit_a, inc=1, device_id=(left,),
                                        device_id_type=pl.DeviceIdType.MESH)
                    pl.semaphore_signal(credit_b, inc=1, device_id=(right,),
                                        device_id_type=pl.DeviceIdType.MESH)
                for j in range(0, n_h, n_q):
                    sl = pl.ds(j, n_q)
                    comm_a[rs, :, sl] = comm_a[rs, :, sl] + tmp_a[:, sl]
                    comm_b[rs, :, sl] = comm_b[rs, :, sl] + tmp_b[:, sl]

            for j in range(0, n_h, n_q):
                sl = pl.ds(j, n_q)
                tmp_a[:, sl] = jnp.maximum(comm_a[1, :, sl], 0.0)
                tmp_b[:, sl] = jnp.maximum(comm_b[1, :, sl], 0.0)
            if p < NP - 1:
                pl.semaphore_signal(credit_a, inc=1, device_id=(left,),
                                    device_id_type=pl.DeviceIdType.MESH)
                pl.semaphore_signal(credit_b, inc=1, device_id=(right,),
                                    device_id_type=pl.DeviceIdType.MESH)
            local_amax = jnp.maximum(local_amax, jnp.max(tmp_a[:, :]))
            local_amax = jnp.maximum(local_amax, jnp.max(tmp_b[:, :]))
            if p < NP - 1:
                st_a = pltpu.make_async_copy(
                    tmp_a, out_ref.at[:, pl.ds(col_a, n_h)], local_sems.at[0])
                st_b = pltpu.make_async_copy(
                    tmp_b, out_ref.at[:, pl.ds(col_b, n_h)], local_sems.at[1])
                st_a.start()
                st_b.start()

        amax_ref[pl.ds(0, 1), :] = jnp.full((1, 128), local_amax, jnp.float32)

        copies = []
        for o in range(1, N_DEV):
            tgt = lax.rem(my + o, N_DEV)
            rd = pltpu.make_async_remote_copy(
                src_ref=amax_ref.at[pl.ds(0, 1)],
                dst_ref=amax_ref.at[pl.ds(o, 1)],
                send_sem=amax_send_sems.at[o],
                recv_sem=amax_recv_sems.at[o],
                device_id=(tgt,),
                device_id_type=pl.DeviceIdType.MESH,
            )
            rd.start()
            copies.append(rd)

        loads = []
        for p in range(NP - 1):
            ld_a = pltpu.make_async_copy(
                out_ref.at[:, pl.ds(p * n_p, n_h)], comm_a.at[p],
                local_sems.at[0])
            ld_b = pltpu.make_async_copy(
                out_ref.at[:, pl.ds(p * n_p + n_h, n_h)], comm_b.at[p],
                local_sems.at[1])
            ld_a.start()
            ld_b.start()
            loads.append((ld_a, ld_b))

        for rd in copies:
            rd.wait()
        gmax = jnp.max(amax_ref[:, :])
        scale = gmax / 127.0

        def quant_inplace(ref, row):
            for j in range(0, n_h, n_h // 2):
                sl = pl.ds(j, n_h // 2)
                if row is None:
                    v = ref[:, sl]
                    ref[:, sl] = jnp.clip(
                        jnp.round(v / scale), -127.0, 127.0) * scale
                else:
                    v = ref[row, :, sl]
                    ref[row, :, sl] = jnp.clip(
                        jnp.round(v / scale), -127.0, 127.0) * scale

        stores = []
        for p in range(NP - 1):
            ld_a, ld_b = loads[p]
            ld_a.wait()
            quant_inplace(comm_a, p)
            so_a = pltpu.make_async_copy(
                comm_a.at[p], out_ref.at[:, pl.ds(p * n_p, n_h)],
                local_sems.at[0])
            so_a.start()
            ld_b.wait()
            quant_inplace(comm_b, p)
            so_b = pltpu.make_async_copy(
                comm_b.at[p], out_ref.at[:, pl.ds(p * n_p + n_h, n_h)],
                local_sems.at[1])
            so_b.start()
            stores.extend((so_a, so_b))
        quant_inplace(tmp_a, None)
        so_a = pltpu.make_async_copy(
            tmp_a, out_ref.at[:, pl.ds((NP - 1) * n_p, n_h)],
            local_sems.at[2])
        so_a.start()
        quant_inplace(tmp_b, None)
        so_b = pltpu.make_async_copy(
            tmp_b, out_ref.at[:, pl.ds((NP - 1) * n_p + n_h, n_h)],
            local_sems.at[3])
        so_b.start()
        stores.extend((so_a, so_b))
        for so in stores:
            so.wait()

    return pl.pallas_call(
        body,
        out_shape=jax.ShapeDtypeStruct((m_per, n), jnp.float32),
        in_specs=[pl.BlockSpec(memory_space=pltpu.VMEM),
                  pl.BlockSpec(memory_space=pltpu.VMEM)],
        out_specs=pl.BlockSpec(memory_space=pl.ANY),
        scratch_shapes=[
            pltpu.VMEM((2, m_per, n_h), jnp.float32),
            pltpu.VMEM((2, m_per, n_h), jnp.float32),
            pltpu.VMEM((m_per, n_h), jnp.float32),
            pltpu.VMEM((m_per, n_h), jnp.float32),
            pltpu.VMEM((N_DEV, 128), jnp.float32),
            pltpu.SemaphoreType.DMA((2,)),
            pltpu.SemaphoreType.DMA((2,)),
            pltpu.SemaphoreType.DMA((2,)),
            pltpu.SemaphoreType.DMA((2,)),
            pltpu.SemaphoreType.DMA((4,)),
            pltpu.SemaphoreType.DMA((4,)),
            pltpu.SemaphoreType.DMA((N_DEV,)),
            pltpu.SemaphoreType.DMA((N_DEV,)),
            pltpu.SemaphoreType.REGULAR,
            pltpu.SemaphoreType.REGULAR,
            pltpu.SemaphoreType.DMA((4,)),
        ],
        compiler_params=pltpu.CompilerParams(
            collective_id=0, vmem_limit_bytes=43 * 1024 * 1024),
    )(x, w_mat)


# device time: 401850 ns/iter; 3.4328x vs baseline; 1.7844x over previous
import jax
import jax.numpy as jnp
from jax import lax
from jax.experimental import pallas as pl
from jax.experimental.pallas import tpu as pltpu

N_DEV = 8
NP = 2


def kernel(x, w_mat):
    m, k_per = x.shape
    _, n = w_mat.shape
    m_per = m // N_DEV
    n_p = n // NP
    n_h = n_p // 2

    def body(x_ref, w_ref, out_ref, comm_a, comm_b, tmp_a, tmp_b,
             stage_a, stage_b, amax_ref,
             sa_send, sa_recv, sb_send, sb_recv, sub_send, sub_recv,
             amax_send_sems, amax_recv_sems, credit_a, credit_b, local_sems):
        my = lax.axis_index("i")
        left = lax.rem(my + N_DEV - 1, N_DEV)
        right = lax.rem(my + 1, N_DEV)

        barrier_sem = pltpu.get_barrier_semaphore()
        for nbr in (left, right):
            pl.semaphore_signal(barrier_sem, inc=1, device_id=(nbr,),
                                device_id_type=pl.DeviceIdType.MESH)
        pl.semaphore_wait(barrier_sem, 2)

        n_q = n_h // 2

        def part_into(ref, c, col):
            xs = x_ref[pl.ds(c * m_per, m_per), :]
            for j in range(0, n_h, n_q):
                ref[:, pl.ds(j, n_q)] = jnp.dot(
                    xs, w_ref[:, col + j:col + j + n_q],
                    preferred_element_type=jnp.float32)

        local_amax = jnp.float32(0.0)
        st_a = st_b = None

        for p in range(NP):
            col_a = p * n_p
            col_b = p * n_p + n_h

            if p >= 1:
                pl.semaphore_wait(credit_a, 1)
                pl.semaphore_wait(credit_b, 1)
            c_a = lax.rem(my + N_DEV - 1, N_DEV)
            c_b = lax.rem(my + 1, N_DEV)
            xs_a = x_ref[pl.ds(c_a * m_per, m_per), :]
            xs_b = x_ref[pl.ds(c_b * m_per, m_per), :]
            subs = []
            for j in range(2):
                sl = pl.ds(j * n_q, n_q)
                for k, (xs, col, comm, s_send, s_recv, dev) in enumerate([
                        (xs_a, col_a, comm_a, sub_send, sub_recv, right),
                        (xs_b, col_b, comm_b, sub_send, sub_recv, left)]):
                    comm[0, :, sl] = jnp.dot(
                        xs, w_ref[:, col + j * n_q:col + (j + 1) * n_q],
                        preferred_element_type=jnp.float32,
                    ).astype(jnp.bfloat16)
                    rd = pltpu.make_async_remote_copy(
                        src_ref=comm.at[0, :, sl], dst_ref=comm.at[1, :, sl],
                        send_sem=s_send.at[2 * j + k],
                        recv_sem=s_recv.at[2 * j + k],
                        device_id=(dev,),
                        device_id_type=pl.DeviceIdType.MESH)
                    rd.start()
                    subs.append(rd)
            if st_a is not None:
                st_a.wait()
                st_b.wait()
            part_into(tmp_a, lax.rem(my + N_DEV - 2, N_DEV), col_a)
            part_into(tmp_b, lax.rem(my + 2, N_DEV), col_b)
            for rd in subs:
                rd.wait()
            pl.semaphore_signal(credit_a, inc=1, device_id=(left,),
                                device_id_type=pl.DeviceIdType.MESH)
            pl.semaphore_signal(credit_b, inc=1, device_id=(right,),
                                device_id_type=pl.DeviceIdType.MESH)
            for j in range(0, n_h, n_q):
                sl = pl.ds(j, n_q)
                comm_a[1, :, sl] = (
                    comm_a[1, :, sl].astype(jnp.float32) + tmp_a[:, sl]
                ).astype(jnp.bfloat16)
                comm_b[1, :, sl] = (
                    comm_b[1, :, sl].astype(jnp.float32) + tmp_b[:, sl]
                ).astype(jnp.bfloat16)

            for s in range(2, N_DEV):
                ss = (s - 1) % 2
                rs = s % 2
                pl.semaphore_wait(credit_a, 1)
                pl.semaphore_wait(credit_b, 1)
                ra = pltpu.make_async_remote_copy(
                    src_ref=comm_a.at[ss], dst_ref=comm_a.at[rs],
                    send_sem=sa_send.at[ss], recv_sem=sa_recv.at[rs],
                    device_id=(right,), device_id_type=pl.DeviceIdType.MESH)
                rb = pltpu.make_async_remote_copy(
                    src_ref=comm_b.at[ss], dst_ref=comm_b.at[rs],
                    send_sem=sb_send.at[ss], recv_sem=sb_recv.at[rs],
                    device_id=(left,), device_id_type=pl.DeviceIdType.MESH)
                ra.start()
                rb.start()
                part_into(tmp_a, lax.rem(my + N_DEV - 1 - s, N_DEV), col_a)
                part_into(tmp_b, lax.rem(my + 1 + s, N_DEV), col_b)
                ra.wait()
                rb.wait()
                if s <= 6:
                    pl.semaphore_signal(credit_a, inc=1, device_id=(left,),
                                        device_id_type=pl.DeviceIdType.MESH)
                    pl.semaphore_signal(credit_b, inc=1, device_id=(right,),
                                        device_id_type=pl.DeviceIdType.MESH)
                for j in range(0, n_h, n_q):
                    sl = pl.ds(j, n_q)
                    if s < N_DEV - 1:
                        comm_a[rs, :, sl] = (
                            comm_a[rs, :, sl].astype(jnp.float32)
                            + tmp_a[:, sl]).astype(jnp.bfloat16)
                        comm_b[rs, :, sl] = (
                            comm_b[rs, :, sl].astype(jnp.float32)
                            + tmp_b[:, sl]).astype(jnp.bfloat16)
                    else:
                        tmp_a[:, sl] = jnp.maximum(
                            comm_a[1, :, sl].astype(jnp.float32)
                            + tmp_a[:, sl], 0.0)
                        tmp_b[:, sl] = jnp.maximum(
                            comm_b[1, :, sl].astype(jnp.float32)
                            + tmp_b[:, sl], 0.0)
            if p < NP - 1:
                pl.semaphore_signal(credit_a, inc=1, device_id=(left,),
                                    device_id_type=pl.DeviceIdType.MESH)
                pl.semaphore_signal(credit_b, inc=1, device_id=(right,),
                                    device_id_type=pl.DeviceIdType.MESH)
            local_amax = jnp.maximum(local_amax, jnp.max(tmp_a[:, :]))
            local_amax = jnp.maximum(local_amax, jnp.max(tmp_b[:, :]))
            if p < NP - 1:
                st_a = pltpu.make_async_copy(
                    tmp_a, out_ref.at[:, pl.ds(col_a, n_h)], local_sems.at[0])
                st_b = pltpu.make_async_copy(
                    tmp_b, out_ref.at[:, pl.ds(col_b, n_h)], local_sems.at[1])
                st_a.start()
                st_b.start()

        amax_ref[pl.ds(0, 1), :] = jnp.full((1, 128), local_amax, jnp.float32)

        copies = []
        for o in range(1, N_DEV):
            tgt = lax.rem(my + o, N_DEV)
            rd = pltpu.make_async_remote_copy(
                src_ref=amax_ref.at[pl.ds(0, 1)],
                dst_ref=amax_ref.at[pl.ds(o, 1)],
                send_sem=amax_send_sems.at[o],
                recv_sem=amax_recv_sems.at[o],
                device_id=(tgt,),
                device_id_type=pl.DeviceIdType.MESH,
            )
            rd.start()
            copies.append(rd)

        loads = []
        for p in range(NP - 1):
            ld_a = pltpu.make_async_copy(
                out_ref.at[:, pl.ds(p * n_p, n_h)], stage_a,
                local_sems.at[0])
            ld_b = pltpu.make_async_copy(
                out_ref.at[:, pl.ds(p * n_p + n_h, n_h)], stage_b,
                local_sems.at[1])
            ld_a.start()
            ld_b.start()
            loads.append((ld_a, ld_b))

        for rd in copies:
            rd.wait()
        gmax = jnp.max(amax_ref[:, :])
        scale = gmax / 127.0

        def quant_inplace(ref, row):
            for j in range(0, n_h, n_h // 2):
                sl = pl.ds(j, n_h // 2)
                if row is None:
                    v = ref[:, sl]
                    ref[:, sl] = jnp.clip(
                        jnp.round(v / scale), -127.0, 127.0) * scale
                else:
                    v = ref[row, :, sl]
                    ref[row, :, sl] = jnp.clip(
                        jnp.round(v / scale), -127.0, 127.0) * scale

        stores = []
        for p in range(NP - 1):
            ld_a, ld_b = loads[p]
            ld_a.wait()
            quant_inplace(stage_a, None)
            so_a = pltpu.make_async_copy(
                stage_a, out_ref.at[:, pl.ds(p * n_p, n_h)],
                local_sems.at[0])
            so_a.start()
            ld_b.wait()
            quant_inplace(stage_b, None)
            so_b = pltpu.make_async_copy(
                stage_b, out_ref.at[:, pl.ds(p * n_p + n_h, n_h)],
                local_sems.at[1])
            so_b.start()
            stores.extend((so_a, so_b))
        quant_inplace(tmp_a, None)
        so_a = pltpu.make_async_copy(
            tmp_a, out_ref.at[:, pl.ds((NP - 1) * n_p, n_h)],
            local_sems.at[2])
        so_a.start()
        quant_inplace(tmp_b, None)
        so_b = pltpu.make_async_copy(
            tmp_b, out_ref.at[:, pl.ds((NP - 1) * n_p + n_h, n_h)],
            local_sems.at[3])
        so_b.start()
        stores.extend((so_a, so_b))
        for so in stores:
            so.wait()

    return pl.pallas_call(
        body,
        out_shape=jax.ShapeDtypeStruct((m_per, n), jnp.float32),
        in_specs=[pl.BlockSpec(memory_space=pltpu.VMEM),
                  pl.BlockSpec(memory_space=pltpu.VMEM)],
        out_specs=pl.BlockSpec(memory_space=pl.ANY),
        scratch_shapes=[
            pltpu.VMEM((2, m_per, n_h), jnp.bfloat16),
            pltpu.VMEM((2, m_per, n_h), jnp.bfloat16),
            pltpu.VMEM((m_per, n_h), jnp.float32),
            pltpu.VMEM((m_per, n_h), jnp.float32),
            pltpu.VMEM((m_per, n_h), jnp.float32),
            pltpu.VMEM((m_per, n_h), jnp.float32),
            pltpu.VMEM((N_DEV, 128), jnp.float32),
            pltpu.SemaphoreType.DMA((2,)),
            pltpu.SemaphoreType.DMA((2,)),
            pltpu.SemaphoreType.DMA((2,)),
            pltpu.SemaphoreType.DMA((2,)),
            pltpu.SemaphoreType.DMA((4,)),
            pltpu.SemaphoreType.DMA((4,)),
            pltpu.SemaphoreType.DMA((N_DEV,)),
            pltpu.SemaphoreType.DMA((N_DEV,)),
            pltpu.SemaphoreType.REGULAR,
            pltpu.SemaphoreType.REGULAR,
            pltpu.SemaphoreType.DMA((4,)),
        ],
        compiler_params=pltpu.CompilerParams(
            collective_id=0, vmem_limit_bytes=43 * 1024 * 1024),
    )(x, w_mat)


# device time: 400721 ns/iter; 3.4425x vs baseline; 1.0028x over previous
import jax
import jax.numpy as jnp
from jax import lax
from jax.experimental import pallas as pl
from jax.experimental.pallas import tpu as pltpu

N_DEV = 8
NP = 2


def kernel(x, w_mat):
    m, k_per = x.shape
    _, n = w_mat.shape
    m_per = m // N_DEV
    n_p = n // NP
    n_h = n_p // 2

    def body(x_ref, w_ref, out_ref, comm_a, comm_b, tmp_a, tmp_b,
             y0_a, y0_b, amax_ref,
             sa_send, sa_recv, sb_send, sb_recv, sub_send, sub_recv,
             amax_send_sems, amax_recv_sems, credit_a, credit_b, local_sems):
        my = lax.axis_index("i")
        left = lax.rem(my + N_DEV - 1, N_DEV)
        right = lax.rem(my + 1, N_DEV)

        barrier_sem = pltpu.get_barrier_semaphore()
        for nbr in (left, right):
            pl.semaphore_signal(barrier_sem, inc=1, device_id=(nbr,),
                                device_id_type=pl.DeviceIdType.MESH)
        pl.semaphore_wait(barrier_sem, 2)

        n_q = n_h // 4

        def part_into(ref, c, col):
            xs = x_ref[pl.ds(c * m_per, m_per), :]
            for j in range(0, n_h, n_q):
                ref[:, pl.ds(j, n_q)] = jnp.dot(
                    xs, w_ref[:, col + j:col + j + n_q],
                    preferred_element_type=jnp.float32)

        local_amax = jnp.float32(0.0)

        for p in range(NP):
            col_a = p * n_p
            col_b = p * n_p + n_h
            dst_a = y0_a if p < NP - 1 else tmp_a
            dst_b = y0_b if p < NP - 1 else tmp_b

            if p >= 1:
                pl.semaphore_wait(credit_a, 1)
                pl.semaphore_wait(credit_b, 1)
            c_a = lax.rem(my + N_DEV - 1, N_DEV)
            c_b = lax.rem(my + 1, N_DEV)
            xs_a = x_ref[pl.ds(c_a * m_per, m_per), :]
            xs_b = x_ref[pl.ds(c_b * m_per, m_per), :]
            subs = []
            for j in range(4):
                sl = pl.ds(j * n_q, n_q)
                for k, (xs, col, comm, dev) in enumerate([
                        (xs_a, col_a, comm_a, right),
                        (xs_b, col_b, comm_b, left)]):
                    comm[0, :, sl] = jnp.dot(
                        xs, w_ref[:, col + j * n_q:col + (j + 1) * n_q],
                        preferred_element_type=jnp.float32,
                    ).astype(jnp.bfloat16)
                    rd = pltpu.make_async_remote_copy(
                        src_ref=comm.at[0, :, sl], dst_ref=comm.at[1, :, sl],
                        send_sem=sub_send.at[2 * j + k],
                        recv_sem=sub_recv.at[2 * j + k],
                        device_id=(dev,),
                        device_id_type=pl.DeviceIdType.MESH)
                    rd.start()
                    subs.append(rd)
            part_into(tmp_a, lax.rem(my + N_DEV - 2, N_DEV), col_a)
            part_into(tmp_b, lax.rem(my + 2, N_DEV), col_b)
            for rd in subs:
                rd.wait()
            pl.semaphore_signal(credit_a, inc=1, device_id=(left,),
                                device_id_type=pl.DeviceIdType.MESH)
            pl.semaphore_signal(credit_b, inc=1, device_id=(right,),
                                device_id_type=pl.DeviceIdType.MESH)
            for j in range(0, n_h, n_q):
                sl = pl.ds(j, n_q)
                comm_a[1, :, sl] = (
                    comm_a[1, :, sl].astype(jnp.float32) + tmp_a[:, sl]
                ).astype(jnp.bfloat16)
                comm_b[1, :, sl] = (
                    comm_b[1, :, sl].astype(jnp.float32) + tmp_b[:, sl]
                ).astype(jnp.bfloat16)

            for s in range(2, N_DEV):
                ss = (s - 1) % 2
                rs = s % 2
                pl.semaphore_wait(credit_a, 1)
                pl.semaphore_wait(credit_b, 1)
                ra = pltpu.make_async_remote_copy(
                    src_ref=comm_a.at[ss], dst_ref=comm_a.at[rs],
                    send_sem=sa_send.at[ss], recv_sem=sa_recv.at[rs],
                    device_id=(right,), device_id_type=pl.DeviceIdType.MESH)
                rb = pltpu.make_async_remote_copy(
                    src_ref=comm_b.at[ss], dst_ref=comm_b.at[rs],
                    send_sem=sb_send.at[ss], recv_sem=sb_recv.at[rs],
                    device_id=(left,), device_id_type=pl.DeviceIdType.MESH)
                ra.start()
                rb.start()
                part_into(tmp_a, lax.rem(my + N_DEV - 1 - s, N_DEV), col_a)
                part_into(tmp_b, lax.rem(my + 1 + s, N_DEV), col_b)
                ra.wait()
                rb.wait()
                if s <= 6:
                    pl.semaphore_signal(credit_a, inc=1, device_id=(left,),
                                        device_id_type=pl.DeviceIdType.MESH)
                    pl.semaphore_signal(credit_b, inc=1, device_id=(right,),
                                        device_id_type=pl.DeviceIdType.MESH)
                for j in range(0, n_h, n_q):
                    sl = pl.ds(j, n_q)
                    if s < N_DEV - 1:
                        comm_a[rs, :, sl] = (
                            comm_a[rs, :, sl].astype(jnp.float32)
                            + tmp_a[:, sl]).astype(jnp.bfloat16)
                        comm_b[rs, :, sl] = (
                            comm_b[rs, :, sl].astype(jnp.float32)
                            + tmp_b[:, sl]).astype(jnp.bfloat16)
                    else:
                        dst_a[:, sl] = jnp.maximum(
                            comm_a[1, :, sl].astype(jnp.float32)
                            + tmp_a[:, sl], 0.0)
                        dst_b[:, sl] = jnp.maximum(
                            comm_b[1, :, sl].astype(jnp.float32)
                            + tmp_b[:, sl], 0.0)

            if p < NP - 1:
                pl.semaphore_signal(credit_a, inc=1, device_id=(left,),
                                    device_id_type=pl.DeviceIdType.MESH)
                pl.semaphore_signal(credit_b, inc=1, device_id=(right,),
                                    device_id_type=pl.DeviceIdType.MESH)
            local_amax = jnp.maximum(local_amax, jnp.max(dst_a[:, :]))
            local_amax = jnp.maximum(local_amax, jnp.max(dst_b[:, :]))

        amax_ref[pl.ds(0, 1), :] = jnp.full((1, 128), local_amax, jnp.float32)

        copies = []
        for o in range(1, N_DEV):
            tgt = lax.rem(my + o, N_DEV)
            rd = pltpu.make_async_remote_copy(
                src_ref=amax_ref.at[pl.ds(0, 1)],
                dst_ref=amax_ref.at[pl.ds(o, 1)],
                send_sem=amax_send_sems.at[o],
                recv_sem=amax_recv_sems.at[o],
                device_id=(tgt,),
                device_id_type=pl.DeviceIdType.MESH,
            )
            rd.start()
            copies.append(rd)
        for rd in copies:
            rd.wait()

        gmax = jnp.max(amax_ref[:, :])
        scale = gmax / 127.0

        def quant_inplace(ref):
            for j in range(0, n_h, n_q):
                sl = pl.ds(j, n_q)
                v = ref[:, sl]
                ref[:, sl] = jnp.clip(
                    jnp.round(v / scale), -127.0, 127.0) * scale

        seg_refs = []
        for p in range(NP - 1):
            seg_refs.extend([(y0_a, p * n_p), (y0_b, p * n_p + n_h)])
        seg_refs.extend([(tmp_a, (NP - 1) * n_p), (tmp_b, (NP - 1) * n_p + n_h)])
        stores = []
        for i, (ref, col) in enumerate(seg_refs):
            quant_inplace(ref)
            so = pltpu.make_async_copy(
                ref, out_ref.at[:, pl.ds(col, n_h)], local_sems.at[i])
            so.start()
            stores.append(so)
        for so in stores:
            so.wait()

    return pl.pallas_call(
        body,
        out_shape=jax.ShapeDtypeStruct((m_per, n), jnp.float32),
        in_specs=[pl.BlockSpec(memory_space=pltpu.VMEM),
                  pl.BlockSpec(memory_space=pltpu.VMEM)],
        out_specs=pl.BlockSpec(memory_space=pl.ANY),
        scratch_shapes=[
            pltpu.VMEM((2, m_per, n_h), jnp.bfloat16),
            pltpu.VMEM((2, m_per, n_h), jnp.bfloat16),
            pltpu.VMEM((m_per, n_h), jnp.float32),
            pltpu.VMEM((m_per, n_h), jnp.float32),
            pltpu.VMEM((m_per, n_h), jnp.float32),
            pltpu.VMEM((m_per, n_h), jnp.float32),
            pltpu.VMEM((N_DEV, 128), jnp.float32),
            pltpu.SemaphoreType.DMA((2,)),
            pltpu.SemaphoreType.DMA((2,)),
            pltpu.SemaphoreType.DMA((2,)),
            pltpu.SemaphoreType.DMA((2,)),
            pltpu.SemaphoreType.DMA((8,)),
            pltpu.SemaphoreType.DMA((8,)),
            pltpu.SemaphoreType.DMA((N_DEV,)),
            pltpu.SemaphoreType.DMA((N_DEV,)),
            pltpu.SemaphoreType.REGULAR,
            pltpu.SemaphoreType.REGULAR,
            pltpu.SemaphoreType.DMA((4,)),
        ],
        compiler_params=pltpu.CompilerParams(
            collective_id=0, vmem_limit_bytes=43 * 1024 * 1024),
    )(x, w_mat)
